# baseline (speedup 1.0000x reference)
import jax
import jax.numpy as jnp
from jax.experimental import pallas as pl
from jax.experimental.pallas import tpu as pltpu

N = 10000
D_IN = 128
DIM = 32
BLK = 400


def _pass1_kernel(A_ref, w_aw_ref, x_ref, w_axw_ref, W_att_ref, y_ref,
                  wcat_ref):
    @pl.when(pl.program_id(0) == 0)
    def _():
        wcat_ref[:, :DIM] = w_aw_ref[:].astype(jnp.bfloat16)
        wcat_ref[:, DIM:] = jnp.dot(
            x_ref[:], w_axw_ref[:],
            preferred_element_type=jnp.float32).astype(jnp.bfloat16)

    h = jnp.dot(A_ref[:].astype(jnp.bfloat16), wcat_ref[:],
                preferred_element_type=jnp.float32)
    a = jnp.maximum(h[:, :DIM], 0.0)
    b = jnp.maximum(h[:, DIM:], 0.0)
    c = (a + b) * 0.5
    context = jnp.tanh(
        jnp.dot(c, W_att_ref[:], preferred_element_type=jnp.float32))
    s1 = jax.nn.sigmoid(jnp.sum(a * context, axis=1, keepdims=True)) + 1e-10
    s2 = jax.nn.sigmoid(jnp.sum(b * context, axis=1, keepdims=True)) + 1e-10
    inv = jax.lax.rsqrt(s1 * s1 + s2 * s2)
    y = a * (s1 * inv) + b * (s2 * inv)
    y_ref[:] = y.astype(jnp.bfloat16)


def _pass2_kernel(A_ref, y_ref, out_ref):
    out_ref[:] = jnp.dot(A_ref[:].astype(jnp.bfloat16), y_ref[:],
                         preferred_element_type=jnp.float32)


def kernel(x, A, w_aw, w_axw, W_att):
    n_blocks = N // BLK

    y = pl.pallas_call(
        _pass1_kernel,
        grid=(n_blocks,),
        in_specs=[
            pl.BlockSpec((BLK, N), lambda i: (i, 0)),
            pl.BlockSpec((N, DIM), lambda i: (0, 0)),
            pl.BlockSpec((N, D_IN), lambda i: (0, 0)),
            pl.BlockSpec((D_IN, DIM), lambda i: (0, 0)),
            pl.BlockSpec((DIM, DIM), lambda i: (0, 0)),
        ],
        out_specs=pl.BlockSpec((BLK, DIM), lambda i: (i, 0)),
        out_shape=jax.ShapeDtypeStruct((N, DIM), jnp.bfloat16),
        scratch_shapes=[pltpu.VMEM((N, 2 * DIM), jnp.bfloat16)],
        compiler_params=pltpu.CompilerParams(
            dimension_semantics=("arbitrary",)),
    )(A, w_aw, x, w_axw, W_att)

    out = pl.pallas_call(
        _pass2_kernel,
        grid=(n_blocks,),
        in_specs=[
            pl.BlockSpec((BLK, N), lambda i: (i, 0)),
            pl.BlockSpec((N, DIM), lambda i: (0, 0)),
        ],
        out_specs=pl.BlockSpec((BLK, DIM), lambda i: (i, 0)),
        out_shape=jax.ShapeDtypeStruct((N, DIM), jnp.float32),
        compiler_params=pltpu.CompilerParams(
            dimension_semantics=("parallel",)),
    )(A, y)

    return out
